# Initial kernel scaffold; baseline (speedup 1.0000x reference)
import jax
import jax.numpy as jnp
from jax.experimental import pallas as pl


def kernel(x, edge_index, edge_attr, batch, W_gin1, b_gin1, W_gin2, b_gin2, Wg1, bg1, Wg2, bg2, Wg3, bg3, Wv1, bv1, Wv2, bv2, Wf1, bf1, Wf2, bf2, Wf3, bf3):
    raise NotImplementedError("write your pallas kernel here")


# speedup vs baseline: 22.3915x; 22.3915x over previous
import functools

import jax
import jax.numpy as jnp
from jax import lax
from jax.experimental import pallas as pl
from jax.experimental.pallas import tpu as pltpu
from jax.experimental.pallas import tpu_sc as plsc

F32 = jnp.float32
NEG = -1e30


_NC = 2
_NS = 16
_CH = 1000


def _sc_edge_agg(table, src, dst, zeros, split_edges):
    n = zeros.shape[0]
    e = src.shape[0]
    _RC = 2000
    n_rc = n // _RC
    assert n % _RC == 0
    rounds = -(-n_rc // _NS)
    if split_edges:
        e_per_w = e // (_NC * _NS)
        assert e % (_NC * _NS) == 0 and e_per_w % _CH == 0
    else:
        e_per_w = e // _NS
        assert e % _NS == 0 and e_per_w % _CH == 0
    n_chunks = e_per_w // _CH

    @functools.partial(
        pl.kernel,
        out_type=jax.ShapeDtypeStruct((_NC, n, 16), F32),
        mesh=plsc.VectorSubcoreMesh(core_axis_name="c", subcore_axis_name="s",
                                    num_cores=_NC, num_subcores=_NS),
        scratch_types=[
            pltpu.VMEM((_CH,), jnp.int32),
            pltpu.VMEM((_CH,), jnp.int32),
            pltpu.VMEM((_CH, 16), F32),
            pltpu.VMEM_SHARED((n, 16), F32),
            pltpu.SemaphoreType.DMA,
        ],
        compiler_params=pltpu.CompilerParams(use_tc_tiling_on_sc=False),
    )
    def k(table_hbm, src_hbm, dst_hbm, zero_hbm, out_hbm, src_v, dst_v,
          rows_v, acc, sem):
        c = lax.axis_index("c")
        s = lax.axis_index("s")
        for j in range(rounds):
            idx = s + _NS * j
            @pl.when(idx < n_rc)
            def _():
                pltpu.sync_copy(zero_hbm.at[pl.ds(idx * _RC, _RC)],
                                acc.at[pl.ds(idx * _RC, _RC)])
        plsc.subcore_barrier()

        if split_edges:
            base0 = (c * _NS + s) * e_per_w
            tbl = table_hbm
        else:
            base0 = s * e_per_w
            tbl = table_hbm.at[c]

        def body(i, carry):
            base = base0 + i * _CH
            pltpu.sync_copy(src_hbm.at[pl.ds(base, _CH)], src_v)
            pltpu.sync_copy(dst_hbm.at[pl.ds(base, _CH)], dst_v)
            pltpu.async_copy(tbl.at[src_v], rows_v, sem).wait()
            pltpu.sync_copy(rows_v, acc.at[dst_v], add=True)
            return carry

        lax.fori_loop(0, n_chunks, body, 0)
        plsc.subcore_barrier()
        for j in range(rounds):
            idx = s + _NS * j
            @pl.when(idx < n_rc)
            def _():
                pltpu.sync_copy(acc.at[pl.ds(idx * _RC, _RC)],
                                out_hbm.at[c, pl.ds(idx * _RC, _RC)])

    return k(table, src, dst, zeros)


_R = 2000


def _dot(a, b):
    return jax.lax.dot_general(
        a, b, (((1,), (0,)), ((), ())), preferred_element_type=F32)


def _dott(a, b):
    return jax.lax.dot_general(
        a, b, (((0,), (0,)), ((), ())),
        precision=jax.lax.Precision.HIGHEST, preferred_element_type=F32)


def _tc_gin1(x, p, w1, b1):
    n = x.shape[0]
    nb = n // _R

    def body(x_ref, p_ref, w_ref, b_ref, out_ref):
        xa = x_ref[...] + p_ref[0] + p_ref[1]
        h = jax.nn.relu(_dot(xa, w_ref[...]) + b_ref[...])
        out_ref[0] = h[:, :16]
        out_ref[1] = h[:, 16:]

    return pl.pallas_call(
        body,
        grid=(nb,),
        in_specs=[
            pl.BlockSpec((_R, 16), lambda i: (i, 0)),
            pl.BlockSpec((2, _R, 16), lambda i: (0, i, 0)),
            pl.BlockSpec((16, 32), lambda i: (0, 0)),
            pl.BlockSpec((1, 32), lambda i: (0, 0)),
        ],
        out_specs=pl.BlockSpec((2, _R, 16), lambda i: (0, i, 0)),
        out_shape=jax.ShapeDtypeStruct((2, n, 16), F32),
    )(x, p, w1, b1)


def _tc_node_mlps(hp, a2, batch_r, g, w2, b2, wg1, bg1, wg2, bg2, wg3, bg3,
                  wv1, bv1, wv2, bv2):
    n = hp.shape[1]
    nb = n // _R

    def body(hp_ref, a2_ref, b_ref, w2_ref, b2_ref, wg1_ref, bg1_ref,
             wg2_ref, bg2_ref, wg3_ref, bg3_ref, wv1_ref, bv1_ref,
             wv2_ref, bv2_ref, gate_ref, val_ref, smax_ref):
        i = pl.program_id(0)
        h = jnp.concatenate([hp_ref[0], hp_ref[1]], axis=1)
        a = jnp.concatenate([a2_ref[0], a2_ref[1]], axis=1)
        h2 = jax.nn.relu(_dot(h + a, w2_ref[...]) + b2_ref[...])
        t = jax.nn.relu(_dot(h2, wg1_ref[...]) + bg1_ref[...])
        t = jax.nn.relu(_dot(t, wg2_ref[...]) + bg2_ref[...])
        gate = _dot(t, wg3_ref[...]) + bg3_ref[...]
        v = jax.nn.relu(_dot(h2, wv1_ref[...]) + bv1_ref[...])
        val = jax.nn.relu(_dot(v, wv2_ref[...]) + bv2_ref[...])
        gate_ref[...] = gate
        val_ref[...] = val
        seg = b_ref[0]
        ids = jax.lax.broadcasted_iota(jnp.int32, (1, g), 1)
        mask = seg == ids
        blkmax = jnp.max(jnp.where(mask, gate, NEG), axis=0, keepdims=True)
        prev = jnp.where(i == 0, jnp.full((1, g), NEG, F32), smax_ref[...])
        smax_ref[...] = jnp.maximum(prev, blkmax)

    wspec = lambda shape: pl.BlockSpec(shape, lambda i: tuple(0 for _ in shape))
    return pl.pallas_call(
        body,
        grid=(nb,),
        in_specs=[
            pl.BlockSpec((2, _R, 16), lambda i: (0, i, 0)),
            pl.BlockSpec((2, _R, 16), lambda i: (0, i, 0)),
            pl.BlockSpec((1, _R, 1), lambda i: (i, 0, 0)),
            wspec((32, 32)), wspec((1, 32)),
            wspec((32, 32)), wspec((1, 32)),
            wspec((32, 32)), wspec((1, 32)),
            wspec((32, 1)), wspec((1, 1)),
            wspec((32, 32)), wspec((1, 32)),
            wspec((32, 32)), wspec((1, 32)),
        ],
        out_specs=[
            pl.BlockSpec((_R, 1), lambda i: (i, 0)),
            pl.BlockSpec((_R, 32), lambda i: (i, 0)),
            pl.BlockSpec((1, g), lambda i: (0, 0)),
        ],
        out_shape=[
            jax.ShapeDtypeStruct((n, 1), F32),
            jax.ShapeDtypeStruct((n, 32), F32),
            jax.ShapeDtypeStruct((1, g), F32),
        ],
    )(hp, a2, batch_r, w2, b2, wg1, bg1, wg2, bg2, wg3, bg3, wv1, bv1,
      wv2, bv2)


def _tc_pool(gate, val, batch_r, smax, g):
    n = gate.shape[0]
    nb = n // _R

    def body(gate_ref, val_ref, b_ref, smax_ref, denom_ref, ev_ref):
        i = pl.program_id(0)
        seg = b_ref[0]
        ids = jax.lax.broadcasted_iota(jnp.int32, (1, g), 1)
        mask = seg == ids
        gmax_row = jnp.max(jnp.where(mask, smax_ref[...], NEG), axis=1,
                           keepdims=True)
        ex = jnp.exp(gate_ref[...] - gmax_row)
        maskf = mask.astype(F32)
        dd = _dott(maskf, ex)
        pv = _dott(maskf, ex * val_ref[...])
        pd = jnp.where(i == 0, jnp.zeros((g, 1), F32), denom_ref[...])
        pe = jnp.where(i == 0, jnp.zeros((g, 32), F32), ev_ref[...])
        denom_ref[...] = pd + dd
        ev_ref[...] = pe + pv

    return pl.pallas_call(
        body,
        grid=(nb,),
        in_specs=[
            pl.BlockSpec((_R, 1), lambda i: (i, 0)),
            pl.BlockSpec((_R, 32), lambda i: (i, 0)),
            pl.BlockSpec((1, _R, 1), lambda i: (i, 0, 0)),
            pl.BlockSpec((1, g), lambda i: (0, 0)),
        ],
        out_specs=[
            pl.BlockSpec((g, 1), lambda i: (0, 0)),
            pl.BlockSpec((g, 32), lambda i: (0, 0)),
        ],
        out_shape=[
            jax.ShapeDtypeStruct((g, 1), F32),
            jax.ShapeDtypeStruct((g, 32), F32),
        ],
    )(gate, val, batch_r, smax)


def _tc_critic(ev, denom, wf1, bf1, wf2, bf2, wf3, bf3):
    g = ev.shape[0]

    def body(ev_ref, d_ref, w1_ref, b1_ref, w2_ref, b2_ref, w3_ref, b3_ref,
             out_ref):
        pooled = ev_ref[...] / (d_ref[...] + 1e-16)
        t = jax.nn.relu(_dot(pooled, w1_ref[...]) + b1_ref[...])
        t = jax.nn.relu(_dot(t, w2_ref[...]) + b2_ref[...])
        out_ref[...] = _dot(t, w3_ref[...]) + b3_ref[...]

    return pl.pallas_call(
        body,
        out_shape=jax.ShapeDtypeStruct((g, 1), F32),
    )(ev, denom, wf1, bf1, wf2, bf2, wf3, bf3)


def kernel(x, edge_index, edge_attr, batch, W_gin1, b_gin1, W_gin2, b_gin2,
           Wg1, bg1, Wg2, bg2, Wg3, bg3, Wv1, bv1, Wv2, bv2,
           Wf1, bf1, Wf2, bf2, Wf3, bf3):
    n, d_in = x.shape
    e = edge_index.shape[1]
    g = 256
    src = edge_index[0]
    dst = edge_index[1]
    zeros = jnp.zeros((n, 16), F32)
    batch_r = batch.reshape(n // _R, _R, 1)

    p = _sc_edge_agg(x, src, dst, zeros, split_edges=True)
    hp = _tc_gin1(x, p, W_gin1, b_gin1.reshape(1, 32))

    a2 = _sc_edge_agg(hp, src, dst, zeros, split_edges=False)

    gate, val, smax = _tc_node_mlps(
        hp, a2, batch_r, g, W_gin2, b_gin2.reshape(1, 32),
        Wg1, bg1.reshape(1, 32), Wg2, bg2.reshape(1, 32),
        Wg3, bg3.reshape(1, 1), Wv1, bv1.reshape(1, 32),
        Wv2, bv2.reshape(1, 32))

    denom, ev = _tc_pool(gate, val, batch_r, smax, g)

    return _tc_critic(ev, denom, Wf1, bf1.reshape(1, 32),
                      Wf2, bf2.reshape(1, 32), Wf3, bf3.reshape(1, 1))


# speedup vs baseline: 36.7714x; 1.6422x over previous
import functools

import jax
import jax.numpy as jnp
from jax import lax
from jax.experimental import pallas as pl
from jax.experimental.pallas import tpu as pltpu
from jax.experimental.pallas import tpu_sc as plsc

F32 = jnp.float32
NEG = -1e30


_NC = 2
_NS = 16
_CH = 200


_NBUF = 5


def _sc_edge_agg(table, ei_r, zeros, split_edges):
    n = zeros.shape[0]
    e = ei_r.shape[1] * _CH
    _RC = 2000
    n_rc = n // _RC
    assert n % _RC == 0
    rounds = -(-n_rc // _NS)
    if split_edges:
        e_per_w = e // (_NC * _NS)
        assert e % (_NC * _NS) == 0
    else:
        e_per_w = e // _NS
        assert e % _NS == 0
    n_chunks = e_per_w // _CH
    assert e_per_w % _CH == 0 and n_chunks % _NBUF == 0
    n_super = n_chunks // _NBUF
    assert n_super % 2 == 0 and n_super >= 4

    @functools.partial(
        pl.kernel,
        out_type=jax.ShapeDtypeStruct((_NC, n, 16), F32),
        mesh=plsc.VectorSubcoreMesh(core_axis_name="c", subcore_axis_name="s",
                                    num_cores=_NC, num_subcores=_NS),
        scratch_types=[
            pltpu.VMEM((_NBUF, 2, 2, _CH), jnp.int32),
            pltpu.VMEM((_NBUF, _CH, 16), F32),
            pltpu.VMEM_SHARED((n, 16), F32),
        ] + [pltpu.SemaphoreType.DMA] * (3 * _NBUF),
        compiler_params=pltpu.CompilerParams(use_tc_tiling_on_sc=False),
    )
    def k(table_hbm, ei_hbm, zero_hbm, out_hbm, idx_v, rows_v, acc, *sems):
        gsem = sems[:_NBUF]
        ssem = sems[_NBUF:2 * _NBUF]
        isem = sems[2 * _NBUF:]
        c = lax.axis_index("c")
        s = lax.axis_index("s")
        for j in range(rounds):
            idx = s + _NS * j
            @pl.when(idx < n_rc)
            def _():
                pltpu.sync_copy(zero_hbm.at[pl.ds(idx * _RC, _RC)],
                                acc.at[pl.ds(idx * _RC, _RC)])
        plsc.subcore_barrier()

        if split_edges:
            cbase0 = (c * _NS + s) * n_chunks
            tbl = table_hbm
        else:
            cbase0 = s * n_chunks
            tbl = table_hbm.at[c]

        def issue_idx(kk, b, par):
            cb = cbase0 + kk * _NBUF + b
            pltpu.async_copy(ei_hbm.at[:, cb], idx_v.at[b, par], isem[b])

        def wait_idx(b, par):
            pltpu.make_async_copy(ei_hbm.at[:, 0], idx_v.at[b, par],
                                  isem[b]).wait()

        def gather(kk_unused, b, par):
            pltpu.async_copy(tbl.at[idx_v.at[b, par, 0]], rows_v.at[b],
                             gsem[b])

        def wait_rows(b, sem):
            pltpu.make_async_copy(zero_hbm.at[pl.ds(0, _CH)], rows_v.at[b],
                                  sem).wait()

        def scatter(b, par):
            pltpu.async_copy(rows_v.at[b], acc.at[idx_v.at[b, par, 1]],
                             ssem[b], add=True)

        for b in range(_NBUF):
            issue_idx(0, b, 0)
        for b in range(_NBUF):
            wait_idx(b, 0)
            gather(0, b, 0)
            issue_idx(1, b, 1)

        def superstep(kk, par):
            for b in range(_NBUF):
                wait_rows(b, gsem[b])
                scatter(b, par)
                wait_idx(b, par ^ 1)
                wait_rows(b, ssem[b])
                gather(kk + 1, b, par ^ 1)
                issue_idx(kk + 2, b, par)

        def pair(m, carry):
            superstep(2 * m, 0)
            superstep(2 * m + 1, 1)
            return carry

        lax.fori_loop(0, (n_super - 2) // 2, pair, 0)
        par = 0
        for b in range(_NBUF):
            wait_rows(b, gsem[b])
            scatter(b, par)
            wait_idx(b, par ^ 1)
            wait_rows(b, ssem[b])
            gather(n_super - 1, b, par ^ 1)
        for b in range(_NBUF):
            wait_rows(b, gsem[b])
            scatter(b, 1)
        for b in range(_NBUF):
            wait_rows(b, ssem[b])
        plsc.subcore_barrier()
        for j in range(rounds):
            idx = s + _NS * j
            @pl.when(idx < n_rc)
            def _():
                pltpu.sync_copy(acc.at[pl.ds(idx * _RC, _RC)],
                                out_hbm.at[c, pl.ds(idx * _RC, _RC)])

    return k(table, ei_r, zeros)


_R = 2000


def _dot(a, b):
    return jax.lax.dot_general(
        a, b, (((1,), (0,)), ((), ())), preferred_element_type=F32)


def _dott(a, b):
    return jax.lax.dot_general(
        a, b, (((0,), (0,)), ((), ())),
        precision=jax.lax.Precision.HIGHEST, preferred_element_type=F32)


def _tc_gin1(x, p, w1, b1):
    n = x.shape[0]
    nb = n // _R

    def body(x_ref, p_ref, w_ref, b_ref, out_ref):
        xa = x_ref[...] + p_ref[0] + p_ref[1]
        h = jax.nn.relu(_dot(xa, w_ref[...]) + b_ref[...])
        out_ref[0] = h[:, :16]
        out_ref[1] = h[:, 16:]

    return pl.pallas_call(
        body,
        grid=(nb,),
        in_specs=[
            pl.BlockSpec((_R, 16), lambda i: (i, 0)),
            pl.BlockSpec((2, _R, 16), lambda i: (0, i, 0)),
            pl.BlockSpec((16, 32), lambda i: (0, 0)),
            pl.BlockSpec((1, 32), lambda i: (0, 0)),
        ],
        out_specs=pl.BlockSpec((2, _R, 16), lambda i: (0, i, 0)),
        out_shape=jax.ShapeDtypeStruct((2, n, 16), F32),
    )(x, p, w1, b1)


def _tc_node_mlps(hp, a2, batch_r, g, w2, b2, wg1, bg1, wg2, bg2, wg3, bg3,
                  wv1, bv1, wv2, bv2):
    n = hp.shape[1]
    nb = n // _R

    def body(hp_ref, a2_ref, b_ref, w2_ref, b2_ref, wg1_ref, bg1_ref,
             wg2_ref, bg2_ref, wg3_ref, bg3_ref, wv1_ref, bv1_ref,
             wv2_ref, bv2_ref, gate_ref, val_ref, smax_ref):
        i = pl.program_id(0)
        h = jnp.concatenate([hp_ref[0], hp_ref[1]], axis=1)
        a = jnp.concatenate([a2_ref[0], a2_ref[1]], axis=1)
        h2 = jax.nn.relu(_dot(h + a, w2_ref[...]) + b2_ref[...])
        t = jax.nn.relu(_dot(h2, wg1_ref[...]) + bg1_ref[...])
        t = jax.nn.relu(_dot(t, wg2_ref[...]) + bg2_ref[...])
        gate = _dot(t, wg3_ref[...]) + bg3_ref[...]
        v = jax.nn.relu(_dot(h2, wv1_ref[...]) + bv1_ref[...])
        val = jax.nn.relu(_dot(v, wv2_ref[...]) + bv2_ref[...])
        gate_ref[...] = gate
        val_ref[...] = val
        seg = b_ref[0]
        ids = jax.lax.broadcasted_iota(jnp.int32, (1, g), 1)
        mask = seg == ids
        blkmax = jnp.max(jnp.where(mask, gate, NEG), axis=0, keepdims=True)
        prev = jnp.where(i == 0, jnp.full((1, g), NEG, F32), smax_ref[...])
        smax_ref[...] = jnp.maximum(prev, blkmax)

    wspec = lambda shape: pl.BlockSpec(shape, lambda i: tuple(0 for _ in shape))
    return pl.pallas_call(
        body,
        grid=(nb,),
        in_specs=[
            pl.BlockSpec((2, _R, 16), lambda i: (0, i, 0)),
            pl.BlockSpec((2, _R, 16), lambda i: (0, i, 0)),
            pl.BlockSpec((1, _R, 1), lambda i: (i, 0, 0)),
            wspec((32, 32)), wspec((1, 32)),
            wspec((32, 32)), wspec((1, 32)),
            wspec((32, 32)), wspec((1, 32)),
            wspec((32, 1)), wspec((1, 1)),
            wspec((32, 32)), wspec((1, 32)),
            wspec((32, 32)), wspec((1, 32)),
        ],
        out_specs=[
            pl.BlockSpec((_R, 1), lambda i: (i, 0)),
            pl.BlockSpec((_R, 32), lambda i: (i, 0)),
            pl.BlockSpec((1, g), lambda i: (0, 0)),
        ],
        out_shape=[
            jax.ShapeDtypeStruct((n, 1), F32),
            jax.ShapeDtypeStruct((n, 32), F32),
            jax.ShapeDtypeStruct((1, g), F32),
        ],
    )(hp, a2, batch_r, w2, b2, wg1, bg1, wg2, bg2, wg3, bg3, wv1, bv1,
      wv2, bv2)


def _tc_pool(gate, val, batch_r, smax, g):
    n = gate.shape[0]
    nb = n // _R

    def body(gate_ref, val_ref, b_ref, smax_ref, denom_ref, ev_ref):
        i = pl.program_id(0)
        seg = b_ref[0]
        ids = jax.lax.broadcasted_iota(jnp.int32, (1, g), 1)
        mask = seg == ids
        gmax_row = jnp.max(jnp.where(mask, smax_ref[...], NEG), axis=1,
                           keepdims=True)
        ex = jnp.exp(gate_ref[...] - gmax_row)
        maskf = mask.astype(F32)
        dd = _dott(maskf, ex)
        pv = _dott(maskf, ex * val_ref[...])
        pd = jnp.where(i == 0, jnp.zeros((g, 1), F32), denom_ref[...])
        pe = jnp.where(i == 0, jnp.zeros((g, 32), F32), ev_ref[...])
        denom_ref[...] = pd + dd
        ev_ref[...] = pe + pv

    return pl.pallas_call(
        body,
        grid=(nb,),
        in_specs=[
            pl.BlockSpec((_R, 1), lambda i: (i, 0)),
            pl.BlockSpec((_R, 32), lambda i: (i, 0)),
            pl.BlockSpec((1, _R, 1), lambda i: (i, 0, 0)),
            pl.BlockSpec((1, g), lambda i: (0, 0)),
        ],
        out_specs=[
            pl.BlockSpec((g, 1), lambda i: (0, 0)),
            pl.BlockSpec((g, 32), lambda i: (0, 0)),
        ],
        out_shape=[
            jax.ShapeDtypeStruct((g, 1), F32),
            jax.ShapeDtypeStruct((g, 32), F32),
        ],
    )(gate, val, batch_r, smax)


def _tc_critic(ev, denom, wf1, bf1, wf2, bf2, wf3, bf3):
    g = ev.shape[0]

    def body(ev_ref, d_ref, w1_ref, b1_ref, w2_ref, b2_ref, w3_ref, b3_ref,
             out_ref):
        pooled = ev_ref[...] / (d_ref[...] + 1e-16)
        t = jax.nn.relu(_dot(pooled, w1_ref[...]) + b1_ref[...])
        t = jax.nn.relu(_dot(t, w2_ref[...]) + b2_ref[...])
        out_ref[...] = _dot(t, w3_ref[...]) + b3_ref[...]

    return pl.pallas_call(
        body,
        out_shape=jax.ShapeDtypeStruct((g, 1), F32),
    )(ev, denom, wf1, bf1, wf2, bf2, wf3, bf3)


def kernel(x, edge_index, edge_attr, batch, W_gin1, b_gin1, W_gin2, b_gin2,
           Wg1, bg1, Wg2, bg2, Wg3, bg3, Wv1, bv1, Wv2, bv2,
           Wf1, bf1, Wf2, bf2, Wf3, bf3):
    n, d_in = x.shape
    e = edge_index.shape[1]
    g = 256
    zeros = jnp.zeros((n, 16), F32)
    batch_r = batch.reshape(n // _R, _R, 1)

    ei_r = edge_index.reshape(2, e // _CH, _CH)

    p = _sc_edge_agg(x, ei_r, zeros, split_edges=True)
    hp = _tc_gin1(x, p, W_gin1, b_gin1.reshape(1, 32))

    a2 = _sc_edge_agg(hp, ei_r, zeros, split_edges=False)

    gate, val, smax = _tc_node_mlps(
        hp, a2, batch_r, g, W_gin2, b_gin2.reshape(1, 32),
        Wg1, bg1.reshape(1, 32), Wg2, bg2.reshape(1, 32),
        Wg3, bg3.reshape(1, 1), Wv1, bv1.reshape(1, 32),
        Wv2, bv2.reshape(1, 32))

    denom, ev = _tc_pool(gate, val, batch_r, smax, g)

    return _tc_critic(ev, denom, Wf1, bf1.reshape(1, 32),
                      Wf2, bf2.reshape(1, 32), Wf3, bf3.reshape(1, 1))
